# Initial kernel scaffold; baseline (speedup 1.0000x reference)
import jax
import jax.numpy as jnp
from jax.experimental import pallas as pl


def kernel(x, ln_g, ln_b, w_qkv, w_light_out, q_routing_token, kv_routing_token, rms_gamma, null_kv, w_q, w_kv, w_heavy_out, null_q_token):
    raise NotImplementedError("write your pallas kernel here")


# speedup vs baseline: 1.3185x; 1.3185x over previous
import functools
import math

import jax
import jax.numpy as jnp
from jax.experimental import pallas as pl

DIM = 768
HEADS = 8
DIM_HEAD = 64
HID = HEADS * DIM_HEAD
WINDOW = 128
NUM_Q = 512
NUM_KV = 512
B = 2
N = 4096
NW = N // WINDOW
EFF_K = min(int(NUM_Q * 9 / 8), N)
NEG = -jnp.finfo(jnp.float32).max


BQ = 1024
WPS = BQ // WINDOW


def _light_body(xp_ref, xc_ref, xn_ref, g_ref, b_ref, wqkv_ref, wo_ref,
                nt_ref, o_ref):
    gidx = pl.program_id(1)
    scale = DIM_HEAD ** -0.5
    xa = jnp.concatenate(
        [xp_ref[0], xc_ref[...].reshape(BQ, DIM), xn_ref[0]], axis=0)
    mu = jnp.mean(xa, axis=-1, keepdims=True)
    var = jnp.mean((xa - mu) ** 2, axis=-1, keepdims=True)
    xan = (xa - mu) / jnp.sqrt(var + 1e-5) * g_ref[...] + b_ref[...]
    q = jnp.dot(xan[WINDOW:WINDOW + BQ], wqkv_ref[:, :HID],
                preferred_element_type=jnp.float32)
    kv = jnp.dot(xan, wqkv_ref[:, HID:], preferred_element_type=jnp.float32)
    jv = jax.lax.broadcasted_iota(jnp.int32, (WINDOW, 3 * WINDOW), 1)
    wrows = []
    for w in range(WPS):
        wg = gidx * WPS + w
        invalid = (((jv < WINDOW) & (wg == 0))
                   | ((jv >= 2 * WINDOW) & (wg == NW - 1)))
        houts = []
        for h in range(HEADS):
            qh = q[w * WINDOW:(w + 1) * WINDOW,
                   h * DIM_HEAD:(h + 1) * DIM_HEAD] * scale
            kh = kv[w * WINDOW:w * WINDOW + 3 * WINDOW,
                    h * DIM_HEAD:(h + 1) * DIM_HEAD]
            vh = kv[w * WINDOW:w * WINDOW + 3 * WINDOW,
                    HID + h * DIM_HEAD:HID + (h + 1) * DIM_HEAD]
            sim = jax.lax.dot_general(qh, kh, (((1,), (1,)), ((), ())),
                                      preferred_element_type=jnp.float32)
            sim = jnp.where(invalid, NEG, sim)
            m = jnp.max(sim, axis=-1, keepdims=True)
            p = jnp.exp(sim - m)
            attn = p / jnp.sum(p, axis=-1, keepdims=True)
            houts.append(jnp.dot(attn, vh, preferred_element_type=jnp.float32))
        wrows.append(jnp.concatenate(houts, axis=1))
    attn_out = jnp.concatenate(wrows, axis=0)
    o_ref[...] = (jnp.dot(attn_out, wo_ref[...],
                          preferred_element_type=jnp.float32) + nt_ref[...])


def _light(x, ln_g, ln_b, w_qkv, w_out, null_tok):
    xw = x.reshape(B * NW, WINDOW, DIM)
    ng = N // BQ

    def im_prev(b, g):
        return (b * NW + jnp.maximum(g * WPS - 1, 0), 0, 0)

    def im_cur(b, g):
        return (b * (NW // WPS) + g, 0, 0)

    def im_nxt(b, g):
        return (b * NW + jnp.minimum((g + 1) * WPS, NW - 1), 0, 0)

    return pl.pallas_call(
        _light_body,
        grid=(B, ng),
        in_specs=[
            pl.BlockSpec((1, WINDOW, DIM), im_prev),
            pl.BlockSpec((WPS, WINDOW, DIM), im_cur),
            pl.BlockSpec((1, WINDOW, DIM), im_nxt),
            pl.BlockSpec((1, DIM), lambda b, g: (0, 0)),
            pl.BlockSpec((1, DIM), lambda b, g: (0, 0)),
            pl.BlockSpec((DIM, 3 * HID), lambda b, g: (0, 0)),
            pl.BlockSpec((HID, DIM), lambda b, g: (0, 0)),
            pl.BlockSpec((1, DIM), lambda b, g: (0, 0)),
        ],
        out_specs=pl.BlockSpec((BQ, DIM), lambda b, g: (b * (N // BQ) + g, 0)),
        out_shape=jax.ShapeDtypeStruct((B * N, DIM), jnp.float32),
    )(xw, xw, xw, ln_g, ln_b, w_qkv, w_out, null_tok)


def _cumsum_lanes(m):
    p = m
    sh = 1
    while sh < m.shape[-1]:
        shifted = jnp.concatenate(
            [jnp.zeros(p.shape[:-1] + (sh,), p.dtype), p[..., :-sh]], axis=-1)
        p = p + shifted
        sh *= 2
    return p


def _route_body(s_ref, idx_ref):
    s = s_ref[...]
    rows = s.shape[0]
    k = NUM_Q
    logk = math.log(float(EFF_K))
    a = jnp.zeros((rows, 1), jnp.float32)
    bneg = -s
    cur = max(4.0, 0.03)
    for _ in range(20):
        sb = (s + bneg) / cur
        amax = jnp.max(sb, axis=-1, keepdims=True)
        lse = jnp.log(jnp.sum(jnp.exp(sb - amax), axis=-1, keepdims=True)) + amax
        a = cur * (logk - lse)
        bneg = -jnp.maximum(s + a, 0.0)
        cur = max(cur * 0.7, 0.03)
    scores = jnp.exp((s + a + bneg) / cur)

    lo = jnp.zeros((rows, 1), jnp.float32)
    hi = jnp.full((rows, 1), 1.1, jnp.float32)
    for _ in range(40):
        mid = (lo + hi) * 0.5
        cnt = jnp.sum((scores >= mid).astype(jnp.float32), axis=-1, keepdims=True)
        pred = cnt >= k
        lo = jnp.where(pred, mid, lo)
        hi = jnp.where(pred, hi, mid)
    t = jnp.max(jnp.where(scores < hi, scores, -1.0), axis=-1, keepdims=True)

    gt = (scores > t).astype(jnp.float32)
    eq = (scores == t).astype(jnp.float32)
    need = k - jnp.sum(gt, axis=-1, keepdims=True)
    eqp = _cumsum_lanes(eq)
    sel = gt + eq * (eqp <= need).astype(jnp.float32)
    prefix = _cumsum_lanes(sel)

    jcol = jax.lax.broadcasted_iota(jnp.int32, (k, 1), 0).astype(jnp.float32)
    out_rows = []
    for r in range(rows):
        pr = prefix[r:r + 1, :]
        cmp = (pr <= jcol).astype(jnp.float32)
        out_rows.append(jnp.sum(cmp, axis=-1, keepdims=True))
    idx = jnp.concatenate(out_rows, axis=1)
    idx_ref[...] = idx.T.astype(jnp.int32)


def _route(s_all):
    return pl.pallas_call(
        _route_body,
        out_shape=jax.ShapeDtypeStruct((2 * B, NUM_Q), jnp.int32),
    )(s_all)


def _gather_body(x_ref, idx_ref, o_ref):
    xb = x_ref[0]
    idxr = idx_ref[0]
    iota = jax.lax.broadcasted_iota(jnp.int32, (N, NUM_Q), 0)
    ohT = (iota == idxr).astype(jnp.float32)
    o_ref[0] = jax.lax.dot_general(ohT, xb, (((0,), (0,)), ((), ())),
                                   preferred_element_type=jnp.float32)


def _gather(x3, idx3):
    return pl.pallas_call(
        _gather_body,
        grid=(2 * B,),
        in_specs=[
            pl.BlockSpec((1, N, DIM), lambda r: (r % B, 0, 0)),
            pl.BlockSpec((1, 1, NUM_Q), lambda r: (r, 0, 0)),
        ],
        out_specs=pl.BlockSpec((1, NUM_Q, DIM), lambda r: (r, 0, 0)),
        out_shape=jax.ShapeDtypeStruct((2 * B, NUM_Q, DIM), jnp.float32),
    )(x3, idx3)


def _heavy_body(tq_ref, tkv_ref, g_ref, nkv_ref, wq_ref, wkv_ref, wo_ref,
                nt_ref, o_ref):
    scale_rms = float(DIM) ** 0.5
    scale = DIM_HEAD ** -0.5

    def rmsn(xb):
        n = jnp.sqrt(jnp.sum(xb * xb, axis=-1, keepdims=True))
        return xb / jnp.maximum(n, 1e-12) * scale_rms * g_ref[...]

    xq = rmsn(tq_ref[0])
    ctx = rmsn(tkv_ref[0])
    q = jnp.dot(xq, wq_ref[...], preferred_element_type=jnp.float32)
    kvm = jnp.dot(ctx, wkv_ref[...], preferred_element_type=jnp.float32)
    nkv = nkv_ref[...]
    outs = []
    for h in range(HEADS):
        qh = q[:, h * DIM_HEAD:(h + 1) * DIM_HEAD]
        kh = kvm[:, h * 2 * DIM_HEAD:h * 2 * DIM_HEAD + DIM_HEAD]
        vh = kvm[:, h * 2 * DIM_HEAD + DIM_HEAD:(h + 1) * 2 * DIM_HEAD]
        kf = jnp.concatenate([nkv[0:1, h * DIM_HEAD:(h + 1) * DIM_HEAD], kh],
                             axis=0)
        vf = jnp.concatenate([nkv[1:2, h * DIM_HEAD:(h + 1) * DIM_HEAD], vh],
                             axis=0)
        sim = jax.lax.dot_general(qh, kf, (((1,), (1,)), ((), ())),
                                  preferred_element_type=jnp.float32) * scale
        m = jnp.max(sim, axis=-1, keepdims=True)
        p = jnp.exp(sim - m)
        attn = p / jnp.sum(p, axis=-1, keepdims=True)
        outs.append(jnp.dot(attn, vf, preferred_element_type=jnp.float32))
    o = jnp.concatenate(outs, axis=1)
    heavy = jnp.dot(o, wo_ref[...], preferred_element_type=jnp.float32)
    o_ref[0] = heavy - nt_ref[...]


def _heavy(toks, rms_gamma, null_kv2, w_q, w_kv, w_out, null_tok):
    return pl.pallas_call(
        _heavy_body,
        grid=(B,),
        in_specs=[
            pl.BlockSpec((1, NUM_Q, DIM), lambda b: (b, 0, 0)),
            pl.BlockSpec((1, NUM_KV, DIM), lambda b: (B + b, 0, 0)),
            pl.BlockSpec((1, DIM), lambda b: (0, 0)),
            pl.BlockSpec((2, HID), lambda b: (0, 0)),
            pl.BlockSpec((DIM, HID), lambda b: (0, 0)),
            pl.BlockSpec((DIM, 2 * HID), lambda b: (0, 0)),
            pl.BlockSpec((HID, DIM), lambda b: (0, 0)),
            pl.BlockSpec((1, DIM), lambda b: (0, 0)),
        ],
        out_specs=pl.BlockSpec((1, NUM_Q, DIM), lambda b: (b, 0, 0)),
        out_shape=jax.ShapeDtypeStruct((B, NUM_Q, DIM), jnp.float32),
    )(toks, toks, rms_gamma, null_kv2, w_q, w_kv, w_out, null_tok)


def _scatter_body(base_ref, idx_ref, hm_ref, o_ref):
    g = pl.program_id(1)
    blk = base_ref.shape[1]
    idxr = idx_ref[0]
    iota = jax.lax.broadcasted_iota(jnp.int32, (blk, NUM_Q), 0) + g * blk
    ohT = (iota == idxr).astype(jnp.float32)
    delta = jnp.dot(ohT, hm_ref[0], preferred_element_type=jnp.float32)
    o_ref[0] = base_ref[0] + delta


def _scatter(base3, idx3, hmn):
    blk = 512
    ng = N // blk
    return pl.pallas_call(
        _scatter_body,
        grid=(B, ng),
        in_specs=[
            pl.BlockSpec((1, blk, DIM), lambda b, g: (b, g, 0)),
            pl.BlockSpec((1, 1, NUM_Q), lambda b, g: (b, 0, 0)),
            pl.BlockSpec((1, NUM_Q, DIM), lambda b, g: (b, 0, 0)),
        ],
        out_specs=pl.BlockSpec((1, blk, DIM), lambda b, g: (b, g, 0)),
        out_shape=jax.ShapeDtypeStruct((B, N, DIM), jnp.float32),
    )(base3, idx3, hmn)


def kernel(x, ln_g, ln_b, w_qkv, w_light_out, q_routing_token,
           kv_routing_token, rms_gamma, null_kv, w_q, w_kv, w_heavy_out,
           null_q_token):
    x2 = x.reshape(B * N, DIM)
    g2 = ln_g.reshape(1, DIM)
    b2 = ln_b.reshape(1, DIM)
    nt2 = null_q_token.reshape(1, DIM)
    gam2 = rms_gamma.reshape(1, DIM)
    nkv2 = null_kv.reshape(2, HID)

    s_q = jnp.einsum('bnd,rd->brn', x, q_routing_token)[:, 0]
    s_kv = jnp.einsum('bnd,rd->brn', x, kv_routing_token)[:, 0]
    s_all = jnp.concatenate([s_q, s_kv], axis=0)

    base = _light(x, g2, b2, w_qkv, w_light_out, nt2)

    idx = _route(s_all)
    idx3 = idx.reshape(2 * B, 1, NUM_Q)
    toks = _gather(x.reshape(B, N, DIM), idx3)
    hmn = _heavy(toks, gam2, nkv2, w_q, w_kv, w_heavy_out, nt2)
    out = _scatter(base.reshape(B, N, DIM), idx3[:B], hmn)
    return out


# speedup vs baseline: 2.0508x; 1.5554x over previous
import functools
import math

import jax
import jax.numpy as jnp
from jax.experimental import pallas as pl
from jax.experimental.pallas import tpu as pltpu
from jax.experimental.pallas import tpu_sc as plsc

DIM = 768
HEADS = 8
DIM_HEAD = 64
HID = HEADS * DIM_HEAD
WINDOW = 128
NUM_Q = 512
NUM_KV = 512
B = 2
N = 4096
NW = N // WINDOW
EFF_K = min(int(NUM_Q * 9 / 8), N)
NEG = -jnp.finfo(jnp.float32).max


BQ = 1024
WPS = BQ // WINDOW


def _light_body(xp_ref, xc_ref, xn_ref, g_ref, b_ref, wqkv_ref, wo_ref,
                nt_ref, o_ref):
    gidx = pl.program_id(1)
    scale = DIM_HEAD ** -0.5
    xa = jnp.concatenate(
        [xp_ref[0], xc_ref[...].reshape(BQ, DIM), xn_ref[0]], axis=0)
    mu = jnp.mean(xa, axis=-1, keepdims=True)
    var = jnp.mean((xa - mu) ** 2, axis=-1, keepdims=True)
    xan = ((xa - mu) / jnp.sqrt(var + 1e-5) * g_ref[...]
           + b_ref[...]).astype(jnp.bfloat16)
    wqkv = wqkv_ref[...].astype(jnp.bfloat16)
    q = jnp.dot(xan[WINDOW:WINDOW + BQ], wqkv[:, :HID],
                preferred_element_type=jnp.float32)
    kv = jnp.dot(xan, wqkv[:, HID:], preferred_element_type=jnp.float32)
    jv = jax.lax.broadcasted_iota(jnp.int32, (1, 3 * WINDOW), 1)
    wrows = []
    for w in range(WPS):
        wg = gidx * WPS + w
        if w == 0:
            bias = jnp.where((jv < WINDOW) & (wg == 0), -1e30, 0.0)
        elif w == WPS - 1:
            bias = jnp.where((jv >= 2 * WINDOW) & (wg == NW - 1), -1e30, 0.0)
        else:
            bias = None
        houts = []
        for h in range(HEADS):
            qh = q[w * WINDOW:(w + 1) * WINDOW,
                   h * DIM_HEAD:(h + 1) * DIM_HEAD] * scale
            kh = kv[w * WINDOW:w * WINDOW + 3 * WINDOW,
                    h * DIM_HEAD:(h + 1) * DIM_HEAD]
            vh = kv[w * WINDOW:w * WINDOW + 3 * WINDOW,
                    HID + h * DIM_HEAD:HID + (h + 1) * DIM_HEAD]
            sim = jax.lax.dot_general(qh, kh, (((1,), (1,)), ((), ())),
                                      preferred_element_type=jnp.float32)
            if bias is not None:
                sim = sim + bias
            p = jnp.exp(sim)
            pv = jnp.dot(p, vh, preferred_element_type=jnp.float32)
            houts.append(pv / jnp.sum(p, axis=-1, keepdims=True))
        wrows.append(jnp.concatenate(houts, axis=1))
    attn_out = jnp.concatenate(wrows, axis=0).astype(jnp.bfloat16)
    o_ref[...] = (jnp.dot(attn_out, wo_ref[...].astype(jnp.bfloat16),
                          preferred_element_type=jnp.float32) + nt_ref[...])


def _light(x, ln_g, ln_b, w_qkv, w_out, null_tok):
    xw = x.reshape(B * NW, WINDOW, DIM)
    ng = N // BQ

    def im_prev(b, g):
        return (b * NW + jnp.maximum(g * WPS - 1, 0), 0, 0)

    def im_cur(b, g):
        return (b * (NW // WPS) + g, 0, 0)

    def im_nxt(b, g):
        return (b * NW + jnp.minimum((g + 1) * WPS, NW - 1), 0, 0)

    return pl.pallas_call(
        _light_body,
        grid=(B, ng),
        in_specs=[
            pl.BlockSpec((1, WINDOW, DIM), im_prev),
            pl.BlockSpec((WPS, WINDOW, DIM), im_cur),
            pl.BlockSpec((1, WINDOW, DIM), im_nxt),
            pl.BlockSpec((1, DIM), lambda b, g: (0, 0)),
            pl.BlockSpec((1, DIM), lambda b, g: (0, 0)),
            pl.BlockSpec((DIM, 3 * HID), lambda b, g: (0, 0)),
            pl.BlockSpec((HID, DIM), lambda b, g: (0, 0)),
            pl.BlockSpec((1, DIM), lambda b, g: (0, 0)),
        ],
        out_specs=pl.BlockSpec((BQ, DIM), lambda b, g: (b * (N // BQ) + g, 0)),
        out_shape=jax.ShapeDtypeStruct((B * N, DIM), jnp.float32),
    )(xw, xw, xw, ln_g, ln_b, w_qkv, w_out, null_tok)


def _cumsum_lanes(m):
    p = m
    sh = 1
    while sh < m.shape[-1]:
        shifted = jnp.concatenate(
            [jnp.zeros(p.shape[:-1] + (sh,), p.dtype), p[..., :-sh]], axis=-1)
        p = p + shifted
        sh *= 2
    return p


def _route_body(s_ref, idx_ref):
    s = s_ref[...]
    rows = s.shape[0]
    k = NUM_Q
    logk = math.log(float(EFF_K))
    a = jnp.zeros((rows, 1), jnp.float32)
    bneg = -s
    cur = max(4.0, 0.03)
    for _ in range(20):
        sb = (s + bneg) / cur
        amax = jnp.max(sb, axis=-1, keepdims=True)
        lse = jnp.log(jnp.sum(jnp.exp(sb - amax), axis=-1, keepdims=True)) + amax
        a = cur * (logk - lse)
        bneg = -jnp.maximum(s + a, 0.0)
        cur = max(cur * 0.7, 0.03)
    scores = jnp.exp((s + a + bneg) / cur)

    lo = jnp.zeros((rows, 1), jnp.float32)
    hi = jnp.full((rows, 1), 1.1, jnp.float32)
    for _ in range(40):
        mid = (lo + hi) * 0.5
        cnt = jnp.sum((scores >= mid).astype(jnp.float32), axis=-1, keepdims=True)
        pred = cnt >= k
        lo = jnp.where(pred, mid, lo)
        hi = jnp.where(pred, hi, mid)
    t = jnp.max(jnp.where(scores < hi, scores, -1.0), axis=-1, keepdims=True)

    gt = (scores > t).astype(jnp.float32)
    eq = (scores == t).astype(jnp.float32)
    need = k - jnp.sum(gt, axis=-1, keepdims=True)
    eqp = _cumsum_lanes(eq)
    sel = gt + eq * (eqp <= need).astype(jnp.float32)
    prefix = _cumsum_lanes(sel)

    jcol = jax.lax.broadcasted_iota(jnp.int32, (k, 1), 0).astype(jnp.float32)
    out_rows = []
    for r in range(rows):
        pr = prefix[r:r + 1, :]
        cmp = (pr <= jcol).astype(jnp.float32)
        out_rows.append(jnp.sum(cmp, axis=-1, keepdims=True))
    idx = jnp.concatenate(out_rows, axis=1)
    idx_ref[...] = idx.T.astype(jnp.int32)


def _route(s_all):
    return pl.pallas_call(
        _route_body,
        out_shape=jax.ShapeDtypeStruct((2 * B, NUM_Q), jnp.int32),
    )(s_all)


def _sc_gather(x2, idx_flat):
    nrows = 2 * B * NUM_Q
    info = plsc.get_sparse_core_info()
    nc, ns = info.num_cores, info.num_subcores
    rows_per = nrows // (nc * ns)
    mesh = plsc.VectorSubcoreMesh(core_axis_name="c", subcore_axis_name="s")

    @functools.partial(
        pl.kernel, mesh=mesh,
        out_type=jax.ShapeDtypeStruct((nrows, DIM), jnp.float32),
        scratch_types=[
            pltpu.VMEM((rows_per,), jnp.int32),
            pltpu.VMEM((rows_per, DIM), jnp.float32),
            pltpu.SemaphoreType.DMA,
        ],
    )
    def gk(x_hbm, idx_hbm, out_hbm, idx_v, rows_v, sem):
        wid = jax.lax.axis_index("s") * nc + jax.lax.axis_index("c")
        base = wid * rows_per
        pltpu.sync_copy(idx_hbm.at[pl.ds(base, rows_per)], idx_v)
        pltpu.async_copy(x_hbm.at[idx_v], rows_v, sem).wait()
        pltpu.sync_copy(rows_v, out_hbm.at[pl.ds(base, rows_per)])

    return gk(x2, idx_flat)


def _gather_body(x_ref, idx_ref, o_ref):
    xb = x_ref[0]
    idxr = idx_ref[0]
    iota = jax.lax.broadcasted_iota(jnp.int32, (N, NUM_Q), 0)
    ohT = (iota == idxr).astype(jnp.float32)
    o_ref[0] = jax.lax.dot_general(ohT, xb, (((0,), (0,)), ((), ())),
                                   preferred_element_type=jnp.float32)


def _gather(x3, idx3):
    return pl.pallas_call(
        _gather_body,
        grid=(2 * B,),
        in_specs=[
            pl.BlockSpec((1, N, DIM), lambda r: (r % B, 0, 0)),
            pl.BlockSpec((1, 1, NUM_Q), lambda r: (r, 0, 0)),
        ],
        out_specs=pl.BlockSpec((1, NUM_Q, DIM), lambda r: (r, 0, 0)),
        out_shape=jax.ShapeDtypeStruct((2 * B, NUM_Q, DIM), jnp.float32),
    )(x3, idx3)


def _heavy_body(tq_ref, tkv_ref, g_ref, nkv_ref, wq_ref, wkv_ref, wo_ref,
                nt_ref, o_ref):
    scale_rms = float(DIM) ** 0.5
    scale = DIM_HEAD ** -0.5

    def rmsn(xb):
        n = jnp.sqrt(jnp.sum(xb * xb, axis=-1, keepdims=True))
        return (xb / jnp.maximum(n, 1e-12) * scale_rms
                * g_ref[...]).astype(jnp.bfloat16)

    xq = rmsn(tq_ref[0])
    ctx = rmsn(tkv_ref[0])
    q = jnp.dot(xq, wq_ref[...].astype(jnp.bfloat16),
                preferred_element_type=jnp.float32)
    kvm = jnp.dot(ctx, wkv_ref[...].astype(jnp.bfloat16),
                  preferred_element_type=jnp.float32)
    nkv = nkv_ref[...]
    outs = []
    for h in range(HEADS):
        qh = q[:, h * DIM_HEAD:(h + 1) * DIM_HEAD]
        kh = kvm[:, h * 2 * DIM_HEAD:h * 2 * DIM_HEAD + DIM_HEAD]
        vh = kvm[:, h * 2 * DIM_HEAD + DIM_HEAD:(h + 1) * 2 * DIM_HEAD]
        kf = jnp.concatenate([nkv[0:1, h * DIM_HEAD:(h + 1) * DIM_HEAD], kh],
                             axis=0)
        vf = jnp.concatenate([nkv[1:2, h * DIM_HEAD:(h + 1) * DIM_HEAD], vh],
                             axis=0)
        sim = jax.lax.dot_general(qh, kf, (((1,), (1,)), ((), ())),
                                  preferred_element_type=jnp.float32) * scale
        p = jnp.exp(sim)
        pv = jnp.dot(p, vf, preferred_element_type=jnp.float32)
        outs.append(pv / jnp.sum(p, axis=-1, keepdims=True))
    o = jnp.concatenate(outs, axis=1).astype(jnp.bfloat16)
    heavy = jnp.dot(o, wo_ref[...].astype(jnp.bfloat16),
                    preferred_element_type=jnp.float32)
    o_ref[0] = heavy - nt_ref[...]


def _heavy(toks, rms_gamma, null_kv2, w_q, w_kv, w_out, null_tok):
    return pl.pallas_call(
        _heavy_body,
        grid=(B,),
        in_specs=[
            pl.BlockSpec((1, NUM_Q, DIM), lambda b: (b, 0, 0)),
            pl.BlockSpec((1, NUM_KV, DIM), lambda b: (B + b, 0, 0)),
            pl.BlockSpec((1, DIM), lambda b: (0, 0)),
            pl.BlockSpec((2, HID), lambda b: (0, 0)),
            pl.BlockSpec((DIM, HID), lambda b: (0, 0)),
            pl.BlockSpec((DIM, 2 * HID), lambda b: (0, 0)),
            pl.BlockSpec((HID, DIM), lambda b: (0, 0)),
            pl.BlockSpec((1, DIM), lambda b: (0, 0)),
        ],
        out_specs=pl.BlockSpec((1, NUM_Q, DIM), lambda b: (b, 0, 0)),
        out_shape=jax.ShapeDtypeStruct((B, NUM_Q, DIM), jnp.float32),
    )(toks, toks, rms_gamma, null_kv2, w_q, w_kv, w_out, null_tok)


def _scatter_body(base_ref, idx_ref, hm_ref, o_ref):
    g = pl.program_id(1)
    blk = base_ref.shape[1]
    idxr = idx_ref[0]
    iota = jax.lax.broadcasted_iota(jnp.int32, (blk, NUM_Q), 0) + g * blk
    ohT = (iota == idxr).astype(jnp.bfloat16)
    delta = jnp.dot(ohT, hm_ref[0].astype(jnp.bfloat16),
                    preferred_element_type=jnp.float32)
    o_ref[0] = base_ref[0] + delta


def _scatter(base3, idx3, hmn):
    blk = 512
    ng = N // blk
    return pl.pallas_call(
        _scatter_body,
        grid=(B, ng),
        in_specs=[
            pl.BlockSpec((1, blk, DIM), lambda b, g: (b, g, 0)),
            pl.BlockSpec((1, 1, NUM_Q), lambda b, g: (b, 0, 0)),
            pl.BlockSpec((1, NUM_Q, DIM), lambda b, g: (b, 0, 0)),
        ],
        out_specs=pl.BlockSpec((1, blk, DIM), lambda b, g: (b, g, 0)),
        out_shape=jax.ShapeDtypeStruct((B, N, DIM), jnp.float32),
    )(base3, idx3, hmn)


def kernel(x, ln_g, ln_b, w_qkv, w_light_out, q_routing_token,
           kv_routing_token, rms_gamma, null_kv, w_q, w_kv, w_heavy_out,
           null_q_token):
    x2 = x.reshape(B * N, DIM)
    g2 = ln_g.reshape(1, DIM)
    b2 = ln_b.reshape(1, DIM)
    nt2 = null_q_token.reshape(1, DIM)
    gam2 = rms_gamma.reshape(1, DIM)
    nkv2 = null_kv.reshape(2, HID)

    s_q = jnp.einsum('bnd,rd->brn', x, q_routing_token)[:, 0]
    s_kv = jnp.einsum('bnd,rd->brn', x, kv_routing_token)[:, 0]
    s_all = jnp.concatenate([s_q, s_kv], axis=0)

    base = _light(x, g2, b2, w_qkv, w_light_out, nt2)

    idx = _route(s_all)
    idx3 = idx.reshape(2 * B, 1, NUM_Q)
    boff = jnp.arange(2 * B, dtype=jnp.int32) % B * N
    idx_flat = (idx + boff[:, None]).reshape(2 * B * NUM_Q)
    toks = _sc_gather(x2, idx_flat).reshape(2 * B, NUM_Q, DIM)
    hmn = _heavy(toks, gam2, nkv2, w_q, w_kv, w_heavy_out, nt2)
    out = _scatter(base.reshape(B, N, DIM), idx3[:B], hmn)
    return out
